# Initial kernel scaffold; baseline (speedup 1.0000x reference)
import jax
import jax.numpy as jnp
from jax.experimental import pallas as pl


def kernel(x, edge_index, W1, b1, W2, b2):
    raise NotImplementedError("write your pallas kernel here")


# speedup vs baseline: 43.6530x; 43.6530x over previous
import functools

import jax
import jax.numpy as jnp
from jax import lax
from jax.experimental import pallas as pl
from jax.experimental.pallas import tpu as pltpu
from jax.experimental.pallas import tpu_sc as plsc

_NC = 2
_NS = 16
_CH = 128
_K = 8
_D = 16
_BB = 2048


def _sc_aggregate(e_chunks, np_rows, d):
    chunks_per_tile = e_chunks // (_NC * _NS)
    groups = chunks_per_tile // _K
    rows_pt = np_rows // _NS
    mesh = plsc.VectorSubcoreMesh(core_axis_name="c", subcore_axis_name="s")

    @functools.partial(
        pl.kernel,
        mesh=mesh,
        out_type=jax.ShapeDtypeStruct((_NC, np_rows, d), jnp.float32),
        compiler_params=pltpu.CompilerParams(use_tc_tiling_on_sc=False),
        scratch_types=[
            pltpu.VMEM((_K, _CH), jnp.int32),
            pltpu.VMEM((_K, _CH), jnp.int32),
            pltpu.VMEM((_K * _CH, d), jnp.float32),
            pltpu.VMEM_SHARED((np_rows, d), jnp.float32),
            pltpu.SemaphoreType.DMA,
        ],
    )
    def kern(feat_hbm, srcc_hbm, dstc_hbm, zeros_hbm, out_hbm,
             src_v, dst_v, rows_v, acc, sem):
        c = lax.axis_index("c")
        s = lax.axis_index("s")
        r0 = s * rows_pt
        pltpu.sync_copy(zeros_hbm.at[pl.ds(r0, rows_pt), :],
                        acc.at[pl.ds(r0, rows_pt), :])
        plsc.subcore_barrier()
        tile_chunk0 = (c * _NS + s) * chunks_per_tile

        @pl.loop(0, groups)
        def _(g):
            cb = tile_chunk0 + g * _K
            pltpu.sync_copy(srcc_hbm.at[pl.ds(cb, _K), :], src_v)
            pltpu.sync_copy(dstc_hbm.at[pl.ds(cb, _K), :], dst_v)
            copies = [
                pltpu.async_copy(feat_hbm.at[src_v.at[j]],
                                 rows_v.at[pl.ds(j * _CH, _CH), :], sem)
                for j in range(_K)
            ]
            for cp in copies:
                cp.wait()
            for j in range(_K):
                pltpu.sync_copy(rows_v.at[pl.ds(j * _CH, _CH), :],
                                acc.at[dst_v.at[j]], add=True)

        plsc.subcore_barrier()
        pltpu.sync_copy(acc.at[pl.ds(r0, rows_pt), :],
                        out_hbm.at[c, pl.ds(r0, rows_pt), :])

    return kern


def _sc_degree(e_chunks, np_rows, d):
    chunks_per_tile = e_chunks // (_NC * _NS)
    groups = chunks_per_tile // _K
    rows_pt = np_rows // _NS
    mesh = plsc.VectorSubcoreMesh(core_axis_name="c", subcore_axis_name="s")

    @functools.partial(
        pl.kernel,
        mesh=mesh,
        out_type=jax.ShapeDtypeStruct((_NC, np_rows, d), jnp.float32),
        compiler_params=pltpu.CompilerParams(use_tc_tiling_on_sc=False),
        scratch_types=[
            pltpu.VMEM((_K, _CH), jnp.int32),
            pltpu.VMEM((_CH, d), jnp.float32),
            pltpu.VMEM_SHARED((np_rows, d), jnp.float32),
            pltpu.SemaphoreType.DMA,
        ],
    )
    def kern(ones_hbm, dstc_hbm, zeros_hbm, out_hbm, dst_v, ones_v, acc, sem):
        c = lax.axis_index("c")
        s = lax.axis_index("s")
        r0 = s * rows_pt
        pltpu.sync_copy(ones_hbm, ones_v)
        pltpu.sync_copy(zeros_hbm.at[pl.ds(r0, rows_pt), :],
                        acc.at[pl.ds(r0, rows_pt), :])
        plsc.subcore_barrier()
        tile_chunk0 = (c * _NS + s) * chunks_per_tile

        @pl.loop(0, groups)
        def _(g):
            cb = tile_chunk0 + g * _K
            pltpu.sync_copy(dstc_hbm.at[pl.ds(cb, _K), :], dst_v)
            for j in range(_K):
                pltpu.sync_copy(ones_v, acc.at[dst_v.at[j]], add=True)

        plsc.subcore_barrier()
        pltpu.sync_copy(acc.at[pl.ds(r0, rows_pt), :],
                        out_hbm.at[c, pl.ds(r0, rows_pt), :])

    return kern


def _scale_kernel(np_rows):
    def body(d0, d1, xr, o):
        deg = d0[...][:, :1] + d1[...][:, :1] + 1.0
        o[...] = xr[...] * lax.rsqrt(deg)

    bs = lambda: pl.BlockSpec((_BB, _D), lambda i: (i, 0))
    return pl.pallas_call(
        body,
        grid=(np_rows // _BB,),
        in_specs=[bs(), bs(), bs()],
        out_specs=bs(),
        out_shape=jax.ShapeDtypeStruct((np_rows, _D), jnp.float32),
    )


def _dense_kernel(np_rows, n_real):
    def body(s0, s1, xsr, d0, d1, w1, b1r, w2, o):
        dinv = lax.rsqrt(d0[...][:, :1] + d1[...][:, :1] + 1.0)
        agg = (s0[...] + s1[...] + xsr[...]) * dinv
        h = jnp.dot(agg, w1[...], preferred_element_type=jnp.float32) + b1r[...]
        h = jnp.maximum(h, 0.0)
        z = jnp.dot(h, w2[...], preferred_element_type=jnp.float32) * dinv
        ridx = (lax.broadcasted_iota(jnp.int32, (_BB, 1), 0)
                + pl.program_id(0) * _BB)
        o[...] = jnp.where(ridx < n_real, z, 0.0)

    bs = lambda: pl.BlockSpec((_BB, _D), lambda i: (i, 0))
    return pl.pallas_call(
        body,
        grid=(np_rows // _BB,),
        in_specs=[bs(), bs(), bs(), bs(), bs(),
                  pl.BlockSpec((16, 32), lambda i: (0, 0)),
                  pl.BlockSpec((1, 32), lambda i: (0, 0)),
                  pl.BlockSpec((32, _D), lambda i: (0, 0))],
        out_specs=bs(),
        out_shape=jax.ShapeDtypeStruct((np_rows, _D), jnp.float32),
    )


def _final_kernel(np_rows):
    def body(s0, s1, zr, d0, d1, b2r, o):
        dinv = lax.rsqrt(d0[...][:, :1] + d1[...][:, :1] + 1.0)
        o[...] = (s0[...] + s1[...] + zr[...]) * dinv + b2r[...]

    bs = lambda: pl.BlockSpec((_BB, _D), lambda i: (i, 0))
    return pl.pallas_call(
        body,
        grid=(np_rows // _BB,),
        in_specs=[bs(), bs(), bs(), bs(), bs(),
                  pl.BlockSpec((1, _D), lambda i: (0, 0))],
        out_specs=bs(),
        out_shape=jax.ShapeDtypeStruct((np_rows, _D), jnp.float32),
    )


def kernel(x, edge_index, W1, b1, W2, b2):
    n = x.shape[0]
    e = edge_index.shape[1]
    f_in = x.shape[1]

    group = _NC * _NS * _CH * _K
    e_pad = ((e + group - 1) // group) * group
    e_chunks = e_pad // _CH
    np_rows = ((n + 1 + _BB - 1) // _BB) * _BB

    ei = edge_index.astype(jnp.int32)
    pad = jnp.full((e_pad - e,), n, dtype=jnp.int32)
    srcc = jnp.concatenate([ei[0], pad]).reshape(e_chunks, _CH)
    dstc = jnp.concatenate([ei[1], pad]).reshape(e_chunks, _CH)

    x_p = jnp.zeros((np_rows, f_in), jnp.float32).at[:n].set(x)
    zeros_nd = jnp.zeros((np_rows, _D), jnp.float32)
    ones_ch = jnp.ones((_CH, _D), jnp.float32)
    w2p = jnp.zeros((W2.shape[0], _D), jnp.float32).at[:, :W2.shape[1]].set(W2)
    b1r = b1.reshape(1, -1)
    b2p = jnp.zeros((1, _D), jnp.float32).at[0, :b2.shape[0]].set(b2)

    deg = _sc_degree(e_chunks, np_rows, _D)(ones_ch, dstc, zeros_nd)
    deg0, deg1 = deg[0], deg[1]

    xs = _scale_kernel(np_rows)(deg0, deg1, x_p)
    s1 = _sc_aggregate(e_chunks, np_rows, _D)(xs, srcc, dstc, zeros_nd)
    z = _dense_kernel(np_rows, n)(s1[0], s1[1], xs, deg0, deg1, W1, b1r, w2p)
    s2 = _sc_aggregate(e_chunks, np_rows, _D)(z, srcc, dstc, zeros_nd)
    outp = _final_kernel(np_rows)(s2[0], s2[1], z, deg0, deg1, b2p)
    return outp[:n, :W2.shape[1]]


# speedup vs baseline: 51.9756x; 1.1907x over previous
import functools

import jax
import jax.numpy as jnp
from jax import lax
from jax.experimental import pallas as pl
from jax.experimental.pallas import tpu as pltpu
from jax.experimental.pallas import tpu_sc as plsc

_NC = 2
_NS = 16
_CH = 128
_K = 8
_D1 = 16
_DD = 8
_D2 = 8
_BB = 2048

_SC_PARAMS = pltpu.CompilerParams(use_tc_tiling_on_sc=False)


def _sc_aggregate(e_chunks, np_rows, d, k):
    chunks_per_tile = e_chunks // (_NC * _NS)
    groups = chunks_per_tile // k
    half_groups = groups // 2
    rows_pt = np_rows // _NS
    mesh = plsc.VectorSubcoreMesh(core_axis_name="c", subcore_axis_name="s")

    @functools.partial(
        pl.kernel,
        mesh=mesh,
        out_type=jax.ShapeDtypeStruct((_NC, np_rows, d), jnp.float32),
        compiler_params=_SC_PARAMS,
        scratch_types=[
            pltpu.VMEM((k, _CH), jnp.int32),
            pltpu.VMEM((k, _CH), jnp.int32),
            pltpu.VMEM((k, _CH), jnp.int32),
            pltpu.VMEM((k, _CH), jnp.int32),
            pltpu.VMEM((k * _CH, d), jnp.float32),
            pltpu.VMEM((k * _CH, d), jnp.float32),
            pltpu.VMEM_SHARED((np_rows, d), jnp.float32),
            pltpu.SemaphoreType.DMA,
            pltpu.SemaphoreType.DMA,
        ],
    )
    def kern(feat_hbm, srcc_hbm, dstc_hbm, zeros_hbm, out_hbm,
             src_a, src_b, dst_a, dst_b, rows_a, rows_b, acc, sem_a, sem_b):
        c = lax.axis_index("c")
        s = lax.axis_index("s")
        r0 = s * rows_pt
        pltpu.sync_copy(zeros_hbm.at[pl.ds(r0, rows_pt), :],
                        acc.at[pl.ds(r0, rows_pt), :])
        plsc.subcore_barrier()
        tile_chunk0 = (c * _NS + s) * chunks_per_tile

        def load_and_fire(g, src_v, dst_v, rows_v, sem):
            cb = tile_chunk0 + g * k
            pltpu.sync_copy(srcc_hbm.at[pl.ds(cb, k), :], src_v)
            pltpu.sync_copy(dstc_hbm.at[pl.ds(cb, k), :], dst_v)
            for j in range(k):
                pltpu.async_copy(feat_hbm.at[src_v.at[j]],
                                 rows_v.at[pl.ds(j * _CH, _CH), :], sem)

        def drain(rows_v, sem):
            pltpu.make_async_copy(feat_hbm.at[pl.ds(0, k * _CH), :],
                                  rows_v, sem).wait()

        def scatter(dst_v, rows_v):
            for j in range(k):
                pltpu.sync_copy(rows_v.at[pl.ds(j * _CH, _CH), :],
                                acc.at[dst_v.at[j]], add=True)

        load_and_fire(0, src_a, dst_a, rows_a, sem_a)

        @pl.loop(0, half_groups)
        def _(gg):
            g1 = 2 * gg + 1
            g2 = 2 * gg + 2
            load_and_fire(g1, src_b, dst_b, rows_b, sem_b)
            drain(rows_a, sem_a)
            scatter(dst_a, rows_a)

            @pl.when(g2 < groups)
            def _():
                load_and_fire(g2, src_a, dst_a, rows_a, sem_a)

            drain(rows_b, sem_b)
            scatter(dst_b, rows_b)

        plsc.subcore_barrier()
        pltpu.sync_copy(acc.at[pl.ds(r0, rows_pt), :],
                        out_hbm.at[c, pl.ds(r0, rows_pt), :])

    return kern


def _sc_degree(e_chunks, np_rows, d):
    chunks_per_tile = e_chunks // (_NC * _NS)
    groups = chunks_per_tile // _K
    half_groups = groups // 2
    rows_pt = np_rows // _NS
    mesh = plsc.VectorSubcoreMesh(core_axis_name="c", subcore_axis_name="s")

    @functools.partial(
        pl.kernel,
        mesh=mesh,
        out_type=jax.ShapeDtypeStruct((_NC, np_rows, d), jnp.float32),
        compiler_params=_SC_PARAMS,
        scratch_types=[
            pltpu.VMEM((_K, _CH), jnp.int32),
            pltpu.VMEM((_K, _CH), jnp.int32),
            pltpu.VMEM((_CH, d), jnp.float32),
            pltpu.VMEM_SHARED((np_rows, d), jnp.float32),
            pltpu.SemaphoreType.DMA,
        ],
    )
    def kern(ones_hbm, dstc_hbm, zeros_hbm, out_hbm,
             dst_a, dst_b, ones_v, acc, sem):
        c = lax.axis_index("c")
        s = lax.axis_index("s")
        r0 = s * rows_pt
        pltpu.sync_copy(ones_hbm, ones_v)
        pltpu.sync_copy(zeros_hbm.at[pl.ds(r0, rows_pt), :],
                        acc.at[pl.ds(r0, rows_pt), :])
        plsc.subcore_barrier()
        tile_chunk0 = (c * _NS + s) * chunks_per_tile

        def scatter(dst_v):
            for j in range(_K):
                pltpu.sync_copy(ones_v, acc.at[dst_v.at[j]], add=True)

        pltpu.sync_copy(dstc_hbm.at[pl.ds(tile_chunk0, _K), :], dst_a)

        @pl.loop(0, half_groups)
        def _(gg):
            g1 = 2 * gg + 1
            g2 = 2 * gg + 2
            cb1 = tile_chunk0 + g1 * _K
            pltpu.sync_copy(dstc_hbm.at[pl.ds(cb1, _K), :], dst_b)
            scatter(dst_a)

            @pl.when(g2 < groups)
            def _():
                cb2 = tile_chunk0 + g2 * _K
                pltpu.sync_copy(dstc_hbm.at[pl.ds(cb2, _K), :], dst_a)

            scatter(dst_b)

        plsc.subcore_barrier()
        pltpu.sync_copy(acc.at[pl.ds(r0, rows_pt), :],
                        out_hbm.at[c, pl.ds(r0, rows_pt), :])

    return kern


def _scale_kernel(np_rows):
    def body(d0, d1, xr, o):
        deg = d0[...][:, :1] + d1[...][:, :1] + 1.0
        o[...] = xr[...] * lax.rsqrt(deg)

    bsd = lambda: pl.BlockSpec((_BB, _DD), lambda i: (i, 0))
    bsx = lambda: pl.BlockSpec((_BB, _D1), lambda i: (i, 0))
    return pl.pallas_call(
        body,
        grid=(np_rows // _BB,),
        in_specs=[bsd(), bsd(), bsx()],
        out_specs=bsx(),
        out_shape=jax.ShapeDtypeStruct((np_rows, _D1), jnp.float32),
    )


def _dense_kernel(np_rows, n_real):
    def body(s0, s1, xsr, d0, d1, w1, b1r, w2, o):
        dinv = lax.rsqrt(d0[...][:, :1] + d1[...][:, :1] + 1.0)
        agg = (s0[...] + s1[...] + xsr[...]) * dinv
        h = jnp.dot(agg, w1[...], preferred_element_type=jnp.float32) + b1r[...]
        h = jnp.maximum(h, 0.0)
        z = jnp.dot(h, w2[...], preferred_element_type=jnp.float32) * dinv
        ridx = (lax.broadcasted_iota(jnp.int32, (_BB, 1), 0)
                + pl.program_id(0) * _BB)
        o[...] = jnp.where(ridx < n_real, z, 0.0)

    bsd = lambda: pl.BlockSpec((_BB, _DD), lambda i: (i, 0))
    bsx = lambda: pl.BlockSpec((_BB, _D1), lambda i: (i, 0))
    bsz = lambda: pl.BlockSpec((_BB, _D2), lambda i: (i, 0))
    return pl.pallas_call(
        body,
        grid=(np_rows // _BB,),
        in_specs=[bsx(), bsx(), bsx(), bsd(), bsd(),
                  pl.BlockSpec((16, 32), lambda i: (0, 0)),
                  pl.BlockSpec((1, 32), lambda i: (0, 0)),
                  pl.BlockSpec((32, _D2), lambda i: (0, 0))],
        out_specs=bsz(),
        out_shape=jax.ShapeDtypeStruct((np_rows, _D2), jnp.float32),
    )


def _final_kernel(np_rows):
    def body(s0, s1, zr, d0, d1, b2r, o):
        dinv = lax.rsqrt(d0[...][:, :1] + d1[...][:, :1] + 1.0)
        o[...] = (s0[...] + s1[...] + zr[...]) * dinv + b2r[...]

    bsd = lambda: pl.BlockSpec((_BB, _DD), lambda i: (i, 0))
    bsz = lambda: pl.BlockSpec((_BB, _D2), lambda i: (i, 0))
    return pl.pallas_call(
        body,
        grid=(np_rows // _BB,),
        in_specs=[bsz(), bsz(), bsz(), bsd(), bsd(),
                  pl.BlockSpec((1, _D2), lambda i: (0, 0))],
        out_specs=bsz(),
        out_shape=jax.ShapeDtypeStruct((np_rows, _D2), jnp.float32),
    )


def kernel(x, edge_index, W1, b1, W2, b2):
    n = x.shape[0]
    e = edge_index.shape[1]
    f_in = x.shape[1]

    group = _NC * _NS * _CH * _K * 2
    e_pad = ((e + group - 1) // group) * group
    e_chunks = e_pad // _CH
    np_rows = ((n + 1 + _BB - 1) // _BB) * _BB

    ei = edge_index.astype(jnp.int32)
    pad = jnp.full((e_pad - e,), n, dtype=jnp.int32)
    srcc = jnp.concatenate([ei[0], pad]).reshape(e_chunks, _CH)
    dstc = jnp.concatenate([ei[1], pad]).reshape(e_chunks, _CH)

    x_p = jnp.zeros((np_rows, f_in), jnp.float32).at[:n].set(x)
    zeros_d1 = jnp.zeros((np_rows, _D1), jnp.float32)
    zeros_dd = jnp.zeros((np_rows, _DD), jnp.float32)
    zeros_d2 = jnp.zeros((np_rows, _D2), jnp.float32)
    ones_ch = jnp.ones((_CH, _DD), jnp.float32)
    w2p = jnp.zeros((W2.shape[0], _D2), jnp.float32).at[:, :W2.shape[1]].set(W2)
    b1r = b1.reshape(1, -1)
    b2p = jnp.zeros((1, _D2), jnp.float32).at[0, :b2.shape[0]].set(b2)

    deg = _sc_degree(e_chunks, np_rows, _DD)(ones_ch, dstc, zeros_dd)
    deg0, deg1 = deg[0], deg[1]

    xs = _scale_kernel(np_rows)(deg0, deg1, x_p)
    s1 = _sc_aggregate(e_chunks, np_rows, _D1, 4)(xs, srcc, dstc, zeros_d1)
    z = _dense_kernel(np_rows, n)(s1[0], s1[1], xs, deg0, deg1, W1, b1r, w2p)
    s2 = _sc_aggregate(e_chunks, np_rows, _D2, 8)(z, srcc, dstc, zeros_d2)
    outp = _final_kernel(np_rows)(s2[0], s2[1], z, deg0, deg1, b2p)
    return outp[:n, :W2.shape[1]]


# speedup vs baseline: 77.2612x; 1.4865x over previous
import functools

import jax
import jax.numpy as jnp
from jax import lax
from jax.experimental import pallas as pl
from jax.experimental.pallas import tpu as pltpu
from jax.experimental.pallas import tpu_sc as plsc

_NC = 2
_NS = 16
_CH = 128
_D = 16
_NPW = 8
_BW = 256

_SC_PARAMS = pltpu.CompilerParams(use_tc_tiling_on_sc=False)


def _sc_aggregate(e_chunks, np_rows, k):
    chunks_per_tile = e_chunks // (_NC * _NS)
    groups = chunks_per_tile // k
    half_groups = groups // 2
    rows_pt = np_rows // _NS
    mesh = plsc.VectorSubcoreMesh(core_axis_name="c", subcore_axis_name="s")

    @functools.partial(
        pl.kernel,
        mesh=mesh,
        out_type=jax.ShapeDtypeStruct((_NC, np_rows, _D), jnp.float32),
        compiler_params=_SC_PARAMS,
        scratch_types=[
            pltpu.VMEM((k, _CH), jnp.int32),
            pltpu.VMEM((k, _CH), jnp.int32),
            pltpu.VMEM((k, _CH), jnp.int32),
            pltpu.VMEM((k, _CH), jnp.int32),
            pltpu.VMEM((k * _CH, _D), jnp.float32),
            pltpu.VMEM((k * _CH, _D), jnp.float32),
            pltpu.VMEM_SHARED((np_rows, _D), jnp.float32),
            pltpu.SemaphoreType.DMA,
            pltpu.SemaphoreType.DMA,
            pltpu.SemaphoreType.DMA,
        ],
    )
    def kern(feat_hbm, srcc_hbm, dstc_hbm, zeros_hbm, out_hbm,
             src_a, src_b, dst_a, dst_b, rows_a, rows_b, acc,
             sem_a, sem_b, sem_s):
        c = lax.axis_index("c")
        s = lax.axis_index("s")
        r0 = s * rows_pt
        pltpu.sync_copy(zeros_hbm.at[pl.ds(r0, rows_pt), :],
                        acc.at[pl.ds(r0, rows_pt), :])
        plsc.subcore_barrier()
        tile_chunk0 = (c * _NS + s) * chunks_per_tile

        def load_and_fire(g, src_v, dst_v, rows_v, sem):
            cb = tile_chunk0 + g * k
            pltpu.sync_copy(srcc_hbm.at[pl.ds(cb, k), :], src_v)
            pltpu.sync_copy(dstc_hbm.at[pl.ds(cb, k), :], dst_v)
            for j in range(k):
                pltpu.async_copy(feat_hbm.at[src_v.at[j]],
                                 rows_v.at[pl.ds(j * _CH, _CH), :], sem)

        def drain_gather(rows_v, sem):
            pltpu.make_async_copy(feat_hbm.at[pl.ds(0, k * _CH), :],
                                  rows_v, sem).wait()

        def scatter(dst_v, rows_v):
            hs = [pltpu.async_copy(rows_v.at[pl.ds(j * _CH, _CH), :],
                                   acc.at[dst_v.at[j]], sem_s, add=True)
                  for j in range(k)]
            for h in hs:
                h.wait()

        load_and_fire(0, src_a, dst_a, rows_a, sem_a)

        @pl.loop(0, half_groups)
        def _(gg):
            g1 = 2 * gg + 1
            g2 = 2 * gg + 2
            load_and_fire(g1, src_b, dst_b, rows_b, sem_b)
            drain_gather(rows_a, sem_a)
            scatter(dst_a, rows_a)

            @pl.when(g2 < groups)
            def _():
                load_and_fire(g2, src_a, dst_a, rows_a, sem_a)

            drain_gather(rows_b, sem_b)
            scatter(dst_b, rows_b)

        plsc.subcore_barrier()
        pltpu.sync_copy(acc.at[pl.ds(r0, rows_pt), :],
                        out_hbm.at[c, pl.ds(r0, rows_pt), :])

    return kern


def _sc_degree(e_chunks, np_rows, k):
    chunks_per_tile = e_chunks // (_NC * _NS)
    groups = chunks_per_tile // k
    half_groups = groups // 2
    rows_pt = np_rows // _NS
    mesh = plsc.VectorSubcoreMesh(core_axis_name="c", subcore_axis_name="s")

    @functools.partial(
        pl.kernel,
        mesh=mesh,
        out_type=jax.ShapeDtypeStruct((_NC, np_rows, _D), jnp.float32),
        compiler_params=_SC_PARAMS,
        scratch_types=[
            pltpu.VMEM((k, _CH), jnp.int32),
            pltpu.VMEM((k, _CH), jnp.int32),
            pltpu.VMEM((_CH, _D), jnp.float32),
            pltpu.VMEM_SHARED((np_rows, _D), jnp.float32),
            pltpu.SemaphoreType.DMA,
            pltpu.SemaphoreType.DMA,
        ],
    )
    def kern(ones_hbm, dstc_hbm, zeros_hbm, out_hbm,
             dst_a, dst_b, ones_v, acc, sem_a, sem_b):
        c = lax.axis_index("c")
        s = lax.axis_index("s")
        r0 = s * rows_pt
        pltpu.sync_copy(ones_hbm, ones_v)
        pltpu.sync_copy(zeros_hbm.at[pl.ds(r0, rows_pt), :],
                        acc.at[pl.ds(r0, rows_pt), :])
        plsc.subcore_barrier()
        tile_chunk0 = (c * _NS + s) * chunks_per_tile

        def fire_scatter(dst_v, sem):
            return [pltpu.async_copy(ones_v, acc.at[dst_v.at[j]], sem,
                                     add=True)
                    for j in range(k)]

        pltpu.sync_copy(dstc_hbm.at[pl.ds(tile_chunk0, k), :], dst_a)

        @pl.loop(0, half_groups)
        def _(gg):
            g1 = 2 * gg + 1
            g2 = 2 * gg + 2
            cb1 = tile_chunk0 + g1 * k
            pltpu.sync_copy(dstc_hbm.at[pl.ds(cb1, k), :], dst_b)
            for h in fire_scatter(dst_a, sem_a):
                h.wait()

            @pl.when(g2 < groups)
            def _():
                cb2 = tile_chunk0 + g2 * k
                pltpu.sync_copy(dstc_hbm.at[pl.ds(cb2, k), :], dst_a)

            for h in fire_scatter(dst_b, sem_b):
                h.wait()

        plsc.subcore_barrier()
        pltpu.sync_copy(acc.at[pl.ds(r0, rows_pt), :],
                        out_hbm.at[c, pl.ds(r0, rows_pt), :])

    return kern


def _scale_kernel(nw_rows):
    def body(dg, xr, o):
        dinv = lax.rsqrt(dg[0] + dg[1] + 1.0)
        o[...] = xr[...] * dinv

    bsw = lambda: pl.BlockSpec((_BW, 128), lambda i: (i, 0))
    return pl.pallas_call(
        body,
        grid=(nw_rows // _BW,),
        in_specs=[pl.BlockSpec((2, _BW, 128), lambda i: (0, i, 0)), bsw()],
        out_specs=bsw(),
        out_shape=jax.ShapeDtypeStruct((nw_rows, 128), jnp.float32),
    )


def _dense_kernel(nw_rows, n_real):
    def body(s1, xsr, dg, w1, b1r, w2, o):
        dinv = lax.rsqrt(dg[0] + dg[1] + 1.0)
        agg = (s1[0] + s1[1] + xsr[...]) * dinv
        h = jnp.dot(agg, w1[...], preferred_element_type=jnp.float32) + b1r[...]
        h = jnp.maximum(h, 0.0)
        z = jnp.dot(h, w2[...], preferred_element_type=jnp.float32) * dinv
        wr = (lax.broadcasted_iota(jnp.int32, (_BW, 128), 0)
              + pl.program_id(0) * _BW)
        lane = lax.broadcasted_iota(jnp.int32, (_BW, 128), 1)
        nid = wr * _NPW + lane // _D
        o[...] = jnp.where(nid < n_real, z, 0.0)

    bsw = lambda: pl.BlockSpec((_BW, 128), lambda i: (i, 0))
    bs2 = lambda: pl.BlockSpec((2, _BW, 128), lambda i: (0, i, 0))
    return pl.pallas_call(
        body,
        grid=(nw_rows // _BW,),
        in_specs=[bs2(), bsw(), bs2(),
                  pl.BlockSpec((128, 256), lambda i: (0, 0)),
                  pl.BlockSpec((1, 256), lambda i: (0, 0)),
                  pl.BlockSpec((256, 128), lambda i: (0, 0))],
        out_specs=bsw(),
        out_shape=jax.ShapeDtypeStruct((nw_rows, 128), jnp.float32),
    )


def _final_kernel(nw_rows):
    def body(s2, zr, dg, b2r, o):
        dinv = lax.rsqrt(dg[0] + dg[1] + 1.0)
        o[...] = (s2[0] + s2[1] + zr[...]) * dinv + b2r[...]

    bsw = lambda: pl.BlockSpec((_BW, 128), lambda i: (i, 0))
    bs2 = lambda: pl.BlockSpec((2, _BW, 128), lambda i: (0, i, 0))
    return pl.pallas_call(
        body,
        grid=(nw_rows // _BW,),
        in_specs=[bs2(), bsw(), bs2(),
                  pl.BlockSpec((1, 128), lambda i: (0, 0))],
        out_specs=bsw(),
        out_shape=jax.ShapeDtypeStruct((nw_rows, 128), jnp.float32),
    )


def kernel(x, edge_index, W1, b1, W2, b2):
    n = x.shape[0]
    e = edge_index.shape[1]
    f_in = x.shape[1]
    f_mid = W1.shape[1]
    f_out = W2.shape[1]

    group = _NC * _NS * _CH * 8 * 2
    e_pad = ((e + group - 1) // group) * group
    e_chunks = e_pad // _CH
    nodes_per_blk = _BW * _NPW
    np_rows = ((n + 1 + nodes_per_blk - 1) // nodes_per_blk) * nodes_per_blk
    nw_rows = np_rows * _D // 128

    ei = edge_index.astype(jnp.int32)
    pad = jnp.full((e_pad - e,), n, dtype=jnp.int32)
    srcc = jnp.concatenate([ei[0], pad]).reshape(e_chunks, _CH)
    dstc = jnp.concatenate([ei[1], pad]).reshape(e_chunks, _CH)

    x_p = jnp.zeros((np_rows, _D), jnp.float32).at[:n, :f_in].set(x)
    xw = x_p.reshape(nw_rows, 128)
    zeros_nd = jnp.zeros((np_rows, _D), jnp.float32)
    ones_ch = jnp.ones((_CH, _D), jnp.float32)

    w2p = jnp.zeros((f_mid, _D), jnp.float32).at[:, :f_out].set(W2)
    eye8 = jnp.eye(_NPW, dtype=jnp.float32)
    w1bd = jnp.kron(eye8, W1)
    w2bd = jnp.kron(eye8, w2p)
    b1bd = jnp.tile(b1, _NPW).reshape(1, _NPW * f_mid)
    b2p = jnp.zeros((_D,), jnp.float32).at[:f_out].set(b2)
    b2bd = jnp.tile(b2p, _NPW).reshape(1, 128)

    deg = _sc_degree(e_chunks, np_rows, 8)(ones_ch, dstc, zeros_nd)
    degw = deg.reshape(_NC, nw_rows, 128)

    xsw = _scale_kernel(nw_rows)(degw, xw)
    s1 = _sc_aggregate(e_chunks, np_rows, 4)(
        xsw.reshape(np_rows, _D), srcc, dstc, zeros_nd)
    s1w = s1.reshape(_NC, nw_rows, 128)
    zw = _dense_kernel(nw_rows, n)(s1w, xsw, degw, w1bd, b1bd, w2bd)
    s2 = _sc_aggregate(e_chunks, np_rows, 4)(
        zw.reshape(np_rows, _D), srcc, dstc, zeros_nd)
    s2w = s2.reshape(_NC, nw_rows, 128)
    outw = _final_kernel(nw_rows)(s2w, zw, degw, b2bd)
    return outw.reshape(np_rows, _D)[:n, :f_out]
